# Initial kernel scaffold; baseline (speedup 1.0000x reference)
import jax
import jax.numpy as jnp
from jax.experimental import pallas as pl


def kernel(input_embedding, position_table, segment_table):
    raise NotImplementedError("write your pallas kernel here")


# speedup vs baseline: 1.9005x; 1.9005x over previous
import jax
import jax.numpy as jnp
from jax.experimental import pallas as pl

_SEQ_BLOCK = 512


def _body(inp_ref, pos_ref, seg_ref, out_ref):
    sb = pl.program_id(0)
    base = sb * _SEQ_BLOCK
    seq_len = pl.num_programs(0) * _SEQ_BLOCK
    idx = base + jax.lax.broadcasted_iota(jnp.int32, (_SEQ_BLOCK, 1), 0)
    mask = idx > (seq_len // 2)
    seg = jnp.where(mask, seg_ref[1, :][None, :], seg_ref[0, :][None, :])
    out_ref[...] = inp_ref[...] + (pos_ref[...] + seg)[None]


def kernel(input_embedding, position_table, segment_table):
    B, S, D = input_embedding.shape
    n_seq = S // _SEQ_BLOCK
    return pl.pallas_call(
        _body,
        grid=(n_seq, B),
        in_specs=[
            pl.BlockSpec((1, _SEQ_BLOCK, D), lambda i, j: (j, i, 0)),
            pl.BlockSpec((_SEQ_BLOCK, D), lambda i, j: (i, 0)),
            pl.BlockSpec(segment_table.shape, lambda i, j: (0, 0)),
        ],
        out_specs=pl.BlockSpec((1, _SEQ_BLOCK, D), lambda i, j: (j, i, 0)),
        out_shape=jax.ShapeDtypeStruct((B, S, D), input_embedding.dtype),
    )(input_embedding, position_table[:S], segment_table)


# speedup vs baseline: 2.2392x; 1.1782x over previous
import jax
import jax.numpy as jnp
from jax.experimental import pallas as pl

_SEQ_BLOCK = 2048


def _body(inp_ref, pos_ref, seg_ref, out_ref):
    sb = pl.program_id(0)
    base = sb * _SEQ_BLOCK
    seq_len = pl.num_programs(0) * _SEQ_BLOCK
    idx = base + jax.lax.broadcasted_iota(jnp.int32, (_SEQ_BLOCK, 1), 0)
    mask = idx > (seq_len // 2)
    seg = jnp.where(mask, seg_ref[1, :][None, :], seg_ref[0, :][None, :])
    out_ref[...] = inp_ref[...] + (pos_ref[...] + seg)[None]


def kernel(input_embedding, position_table, segment_table):
    B, S, D = input_embedding.shape
    n_seq = S // _SEQ_BLOCK
    return pl.pallas_call(
        _body,
        grid=(n_seq, B),
        in_specs=[
            pl.BlockSpec((1, _SEQ_BLOCK, D), lambda i, j: (j, i, 0)),
            pl.BlockSpec((_SEQ_BLOCK, D), lambda i, j: (i, 0)),
            pl.BlockSpec(segment_table.shape, lambda i, j: (0, 0)),
        ],
        out_specs=pl.BlockSpec((1, _SEQ_BLOCK, D), lambda i, j: (j, i, 0)),
        out_shape=jax.ShapeDtypeStruct((B, S, D), input_embedding.dtype),
    )(input_embedding, position_table[:S], segment_table)
